# Initial kernel scaffold; baseline (speedup 1.0000x reference)
import jax
import jax.numpy as jnp
from jax.experimental import pallas as pl


def kernel(input, edge_index, W, b):
    raise NotImplementedError("write your pallas kernel here")


# speedup vs baseline: 2.4668x; 2.4668x over previous
import jax
import jax.numpy as jnp
from jax import lax
from jax.experimental import pallas as pl
from jax.experimental.pallas import tpu as pltpu
from jax.experimental.pallas import tpu_sc as plsc

N = 10000
E = 160000
D = 128

NC = 2
NS = 16
L = 16
NW = NC * NS
EPW = 5120
E_PAD = NW * EPW
CH = 256
NCHUNK = EPW // CH
R = 10240
TRASH = N
RPT = R // NS

ROW_BLOCK = 1000


def _support_body(x_ref, w_ref, o_ref):
    o_ref[...] = jnp.dot(x_ref[...], w_ref[...],
                         preferred_element_type=jnp.float32)


def _combine_body(p_ref, b_ref, o_ref):
    o_ref[...] = p_ref[0] + p_ref[1] + b_ref[...]


def _agg_body(skeys, support, zrows, pout, kv, vidx, uidx, rows, acc, sem):
    cid = lax.axis_index("c")
    sid = lax.axis_index("s")
    wid = sid * NC + cid

    pltpu.sync_copy(zrows, acc.at[pl.ds(sid * RPT, RPT)])
    plsc.subcore_barrier()

    base = 8 + wid * EPW

    @pl.loop(0, NCHUNK)
    def _chunk(c):
        goff = base + c * CH
        pltpu.sync_copy(skeys.at[pl.ds(goff - 8, CH + 8)], kv)
        for i in range(CH // L):
            cur = kv[pl.ds(8 + i * L, L)]
            prv = kv[pl.ds(7 + i * L, L)]
            u = lax.div(cur, N)
            v = cur - u * N
            ue = jnp.where(cur == prv, TRASH, u)
            j, col = i // 8, (i % 8) * L
            vidx[j, pl.ds(col, L)] = v
            uidx[j, pl.ds(col, L)] = ue
        cps = [pltpu.async_copy(support.at[vidx.at[j]],
                                rows.at[pl.ds(j * 128, 128)], sem)
               for j in range(CH // 128)]
        for cp in cps:
            cp.wait()
        for j in range(CH // 128):
            pltpu.sync_copy(rows.at[pl.ds(j * 128, 128)],
                            acc.at[uidx.at[j]], add=True)

    plsc.subcore_barrier()
    pltpu.sync_copy(acc.at[pl.ds(sid * RPT, RPT)],
                    pout.at[cid, pl.ds(sid * RPT, RPT)])


def _make_agg():
    mesh = plsc.VectorSubcoreMesh(core_axis_name="c", subcore_axis_name="s",
                                  num_cores=NC, num_subcores=NS)
    return pl.kernel(
        _agg_body,
        out_type=jax.ShapeDtypeStruct((NC, R, D), jnp.float32),
        mesh=mesh,
        scratch_types=[
            pltpu.VMEM((CH + 8,), jnp.int32),
            pltpu.VMEM((CH // 128, 128), jnp.int32),
            pltpu.VMEM((CH // 128, 128), jnp.int32),
            pltpu.VMEM((CH, D), jnp.float32),
            pltpu.VMEM_SHARED((R, D), jnp.float32),
            pltpu.SemaphoreType.DMA,
        ],
    )


def kernel(input, edge_index, W, b):
    x = input
    n = x.shape[0]
    assert n == N and x.shape[1] == D and edge_index.shape == (2, E)

    support = pl.pallas_call(
        _support_body,
        grid=(N // ROW_BLOCK,),
        in_specs=[pl.BlockSpec((ROW_BLOCK, D), lambda i: (i, 0)),
                  pl.BlockSpec((D, D), lambda i: (0, 0))],
        out_specs=pl.BlockSpec((ROW_BLOCK, D), lambda i: (i, 0)),
        out_shape=jax.ShapeDtypeStruct((N, D), jnp.float32),
    )(x, W)

    enc = edge_index[0].astype(jnp.int32) * N + edge_index[1].astype(jnp.int32)
    skeys = jnp.concatenate([
        jnp.full((8,), -1, jnp.int32),
        jnp.sort(enc),
        jnp.full((E_PAD - E,), N * N, jnp.int32),
    ])
    zrows = jnp.zeros((RPT, D), jnp.float32)

    pout = _make_agg()(skeys, support, zrows)

    out = pl.pallas_call(
        _combine_body,
        grid=(N // ROW_BLOCK,),
        in_specs=[pl.BlockSpec((NC, ROW_BLOCK, D), lambda i: (0, i, 0)),
                  pl.BlockSpec((1, D), lambda i: (0, 0))],
        out_specs=pl.BlockSpec((ROW_BLOCK, D), lambda i: (i, 0)),
        out_shape=jax.ShapeDtypeStruct((N, D), jnp.float32),
    )(pout, b.reshape(1, D))
    return out


# speedup vs baseline: 2.4928x; 1.0105x over previous
import jax
import jax.numpy as jnp
from jax import lax
from jax.experimental import pallas as pl
from jax.experimental.pallas import tpu as pltpu
from jax.experimental.pallas import tpu_sc as plsc

N = 10000
E = 160000
D = 128

NC = 2
NS = 16
L = 16
NW = NC * NS
EPW = 5120
E_PAD = NW * EPW
CH = 128
NCHUNK = EPW // CH
R = 10240
TRASH = N
RPT = R // NS

ROW_BLOCK = 1000


def _support_body(x_ref, w_ref, o_ref):
    o_ref[...] = jnp.dot(x_ref[...], w_ref[...],
                         preferred_element_type=jnp.float32)


def _combine_body(p_ref, b_ref, o_ref):
    o_ref[...] = p_ref[0] + p_ref[1] + b_ref[...]


def _agg_body(skeys, support, zrows, pout, kv0, kv1, vidx, uidx, rows, acc,
              sem0, sem1):
    cid = lax.axis_index("c")
    sid = lax.axis_index("s")
    wid = sid * NC + cid
    sems = [sem0, sem1]
    kvs = [kv0, kv1]

    pltpu.sync_copy(zrows, acc.at[pl.ds(sid * RPT, RPT)])
    plsc.subcore_barrier()

    base = 8 + wid * EPW

    def _prep(k, b):
        goff = base + k * CH
        pltpu.sync_copy(skeys.at[pl.ds(goff - 8, CH + 8)], kvs[b])
        for i in range(CH // L):
            cur = kvs[b][pl.ds(8 + i * L, L)]
            prv = kvs[b][pl.ds(7 + i * L, L)]
            u = lax.div(cur, N)
            v = cur - u * N
            ue = jnp.where(cur == prv, TRASH, u)
            vidx[b, pl.ds(i * L, L)] = v
            uidx[b, pl.ds(i * L, L)] = ue
        pltpu.async_copy(support.at[vidx.at[b]],
                         rows.at[pl.ds(b * CH, CH)], sems[b])

    _prep(0, 0)

    @pl.loop(0, NCHUNK, step=2)
    def _chunk(c):
        for b in range(2):
            k = c + b
            pltpu.make_async_copy(support.at[vidx.at[b]],
                                  rows.at[pl.ds(b * CH, CH)], sems[b]).wait()

            @pl.when(k + 1 < NCHUNK)
            def _():
                _prep(k + 1, 1 - b)

            pltpu.sync_copy(rows.at[pl.ds(b * CH, CH)],
                            acc.at[uidx.at[b]], add=True)

    plsc.subcore_barrier()
    pltpu.sync_copy(acc.at[pl.ds(sid * RPT, RPT)],
                    pout.at[cid, pl.ds(sid * RPT, RPT)])


def _make_agg():
    mesh = plsc.VectorSubcoreMesh(core_axis_name="c", subcore_axis_name="s",
                                  num_cores=NC, num_subcores=NS)
    return pl.kernel(
        _agg_body,
        out_type=jax.ShapeDtypeStruct((NC, R, D), jnp.float32),
        mesh=mesh,
        scratch_types=[
            pltpu.VMEM((CH + 8,), jnp.int32),
            pltpu.VMEM((CH + 8,), jnp.int32),
            pltpu.VMEM((2, CH), jnp.int32),
            pltpu.VMEM((2, CH), jnp.int32),
            pltpu.VMEM((2 * CH, D), jnp.float32),
            pltpu.VMEM_SHARED((R, D), jnp.float32),
            pltpu.SemaphoreType.DMA,
            pltpu.SemaphoreType.DMA,
        ],
    )


def kernel(input, edge_index, W, b):
    x = input
    n = x.shape[0]
    assert n == N and x.shape[1] == D and edge_index.shape == (2, E)

    support = pl.pallas_call(
        _support_body,
        grid=(N // ROW_BLOCK,),
        in_specs=[pl.BlockSpec((ROW_BLOCK, D), lambda i: (i, 0)),
                  pl.BlockSpec((D, D), lambda i: (0, 0))],
        out_specs=pl.BlockSpec((ROW_BLOCK, D), lambda i: (i, 0)),
        out_shape=jax.ShapeDtypeStruct((N, D), jnp.float32),
    )(x, W)

    enc = edge_index[0].astype(jnp.int32) * N + edge_index[1].astype(jnp.int32)
    skeys = jnp.concatenate([
        jnp.full((8,), -1, jnp.int32),
        jnp.sort(enc),
        jnp.full((E_PAD - E,), N * N, jnp.int32),
    ])
    zrows = jnp.zeros((RPT, D), jnp.float32)

    pout = _make_agg()(skeys, support, zrows)

    out = pl.pallas_call(
        _combine_body,
        grid=(N // ROW_BLOCK,),
        in_specs=[pl.BlockSpec((NC, ROW_BLOCK, D), lambda i: (0, i, 0)),
                  pl.BlockSpec((1, D), lambda i: (0, 0))],
        out_specs=pl.BlockSpec((ROW_BLOCK, D), lambda i: (i, 0)),
        out_shape=jax.ShapeDtypeStruct((N, D), jnp.float32),
    )(pout, b.reshape(1, D))
    return out
